# baseline (speedup 1.0000x reference)
import functools

import jax
import jax.numpy as jnp
from jax import lax
from jax.experimental import pallas as pl
from jax.experimental.pallas import tpu as pltpu
from jax.experimental.pallas import tpu_sc as plsc

_NC = 2
_NS = 16
_NW = _NC * _NS
_D = 64
_IB = 128
_NB = 4
_CHUNK = _IB * _NB


@functools.cache
def _make_embed(n_rows):
    rows_w = n_rows // _NW
    n_chunks = rows_w // _CHUNK
    blocks_w = rows_w // _IB
    mesh = plsc.VectorSubcoreMesh(core_axis_name="c", subcore_axis_name="s")

    @functools.partial(
        pl.kernel,
        out_type=jax.ShapeDtypeStruct((n_rows, _D), jnp.float32),
        mesh=mesh,
        scratch_types=[
            pltpu.VMEM((_NB, _IB), jnp.int32),
            pltpu.VMEM((_CHUNK, _D), jnp.float32),
            pltpu.SemaphoreType.DMA,
        ],
        compiler_params=pltpu.CompilerParams(use_tc_tiling_on_sc=False),
    )
    def k(x_hbm, table_hbm, out_hbm, idx_v, rows_v, sem):
        wid = lax.axis_index("s") * _NC + lax.axis_index("c")
        base_blk = wid * blocks_w

        def body(c, carry):
            blk = base_blk + c * _NB
            pltpu.sync_copy(x_hbm.at[pl.ds(blk, _NB)], idx_v)
            copies = [
                pltpu.async_copy(table_hbm.at[idx_v.at[j]],
                                 rows_v.at[pl.ds(j * _IB, _IB)], sem)
                for j in range(_NB)
            ]
            for cp in copies:
                cp.wait()
            pltpu.sync_copy(rows_v, out_hbm.at[pl.ds(blk * _IB, _CHUNK)])
            return carry

        lax.fori_loop(0, n_chunks, body, 0)

    return k


def kernel(x, table):
    b, s = x.shape
    n = b * s
    x_blocks = x.reshape(n // _IB, _IB)
    out = _make_embed(n)(x_blocks, table)
    return out.reshape(b, s, _D)


# speedup vs baseline: 1.0376x; 1.0376x over previous
import functools

import jax
import jax.numpy as jnp
from jax import lax
from jax.experimental import pallas as pl
from jax.experimental.pallas import tpu as pltpu
from jax.experimental.pallas import tpu_sc as plsc

_NC = 2
_NS = 16
_NW = _NC * _NS
_D = 64
_IB = 128
_NB = 4
_CHUNK = _IB * _NB


@functools.cache
def _make_embed(n_rows):
    rows_w = n_rows // _NW
    blocks_w = rows_w // _IB
    n_pairs = rows_w // (2 * _CHUNK)
    mesh = plsc.VectorSubcoreMesh(core_axis_name="c", subcore_axis_name="s")

    @functools.partial(
        pl.kernel,
        out_type=jax.ShapeDtypeStruct((n_rows, _D), jnp.float32),
        mesh=mesh,
        scratch_types=[
            pltpu.VMEM((blocks_w, _IB), jnp.int32),
            pltpu.VMEM((_CHUNK, _D), jnp.float32),
            pltpu.VMEM((_CHUNK, _D), jnp.float32),
            pltpu.SemaphoreType.DMA,
            pltpu.SemaphoreType.DMA,
            pltpu.SemaphoreType.DMA,
            pltpu.SemaphoreType.DMA,
        ],
        compiler_params=pltpu.CompilerParams(use_tc_tiling_on_sc=False),
    )
    def k(x_hbm, table_hbm, out_hbm, idx_v, rows0, rows1, g0, g1, o0, o1):
        rows = (rows0, rows1)
        gsem = (g0, g1)
        osem = (o0, o1)
        wid = lax.axis_index("s") * _NC + lax.axis_index("c")
        base_blk = wid * blocks_w
        pltpu.sync_copy(x_hbm.at[pl.ds(base_blk, blocks_w)], idx_v)

        def drain_out(b):
            pltpu.make_async_copy(
                rows[b], out_hbm.at[pl.ds(0, _CHUNK)], osem[b]).wait()

        def drain_gather(b):
            pltpu.make_async_copy(
                table_hbm.at[pl.ds(0, _CHUNK)], rows[b], gsem[b]).wait()

        def body(i, carry):
            for b in range(2):
                lblk = (2 * i + b) * _NB

                @pl.when(i > 0)
                def _(b=b):
                    drain_out(b)

                for j in range(_NB):
                    pltpu.async_copy(
                        table_hbm.at[idx_v.at[lblk + j]],
                        rows[b].at[pl.ds(j * _IB, _IB)], gsem[b])
            for b in range(2):
                lblk = (2 * i + b) * _NB
                drain_gather(b)
                pltpu.async_copy(
                    rows[b],
                    out_hbm.at[pl.ds((base_blk + lblk) * _IB, _CHUNK)],
                    osem[b])
            return carry

        lax.fori_loop(0, n_pairs, body, 0)
        for b in range(2):
            drain_out(b)

    return k


def kernel(x, table):
    b, s = x.shape
    n = b * s
    x_blocks = x.reshape(n // _IB, _IB)
    out = _make_embed(n)(x_blocks, table)
    return out.reshape(b, s, _D)
